# baseline (speedup 1.0000x reference)
import functools

import jax
import jax.numpy as jnp
from jax import lax
from jax.experimental import pallas as pl
from jax.experimental.pallas import tpu as pltpu
from jax.experimental.pallas import tpu_sc as plsc

D = 64
SCALE = 8.0
B = 4096
L = 200
V = 1000000
BB = 128
VB = 2048
NG = (V + VB - 1) // VB
VP = NG * (VB // 2)

_info = plsc.get_sparse_core_info()
NC, NS, LN = _info.num_cores, _info.num_subcores, _info.num_lanes
NW = NC * NS

_mesh = plsc.VectorSubcoreMesh(core_axis_name="c", subcore_axis_name="s")


def _pack(t):
    def body(i_ref, o_ref):
        t = i_ref[...]
        o_ref[...] = jnp.concatenate([t[: VB // 2], t[VB // 2 :]], axis=1)

    return pl.pallas_call(
        body,
        grid=(NG,),
        in_specs=[pl.BlockSpec((VB, D), lambda g: (g, 0))],
        out_specs=pl.BlockSpec((VB // 2, 2 * D), lambda g: (g, 0)),
        out_shape=jax.ShapeDtypeStruct((VP, 2 * D), jnp.float32),
    )(t)


@functools.partial(
    pl.kernel,
    mesh=_mesh,
    compiler_params=pltpu.CompilerParams(
        use_tc_tiling_on_sc=False, needs_layout_passes=False),
    out_type=jax.ShapeDtypeStruct((L * D, B), jnp.float32),
    scratch_types=[
        pltpu.VMEM((L, BB), jnp.int32),
        pltpu.VMEM((L, BB), jnp.int32),
        pltpu.VMEM((BB, 2 * D), jnp.float32),
        pltpu.VMEM((BB, 2 * D), jnp.float32),
        pltpu.VMEM((D, BB), jnp.float32),
        pltpu.VMEM((D, BB), jnp.float32),
        pltpu.SemaphoreType.DMA,
        pltpu.SemaphoreType.DMA,
        pltpu.SemaphoreType.DMA,
        pltpu.SemaphoreType.DMA,
    ],
)
def _gather_scale_t(xt_hbm, table_hbm, out_hbm,
                    idx_v, idxg_v, rows_a, rows_b, tr_a, tr_b,
                    ga, gb, sta, stb):
    wid = lax.axis_index("s") * NC + lax.axis_index("c")
    rows = (rows_a, rows_b)
    trs = (tr_a, tr_b)
    gsem = (ga, gb)
    ssem = (sta, stb)

    pltpu.sync_copy(xt_hbm.at[:, pl.ds(wid * BB, BB)], idx_v)

    @plsc.parallel_loop(0, L, step=1, unroll=2)
    def _shift(li):
        for c in range(BB // LN):
            sl = pl.ds(c * LN, LN)
            xi = idx_v[li, sl]
            idxg_v[li, sl] = (
                lax.shift_left(lax.shift_right_logical(xi, 11), 10)
                + jnp.bitwise_and(xi, VB // 2 - 1))

    def gath(li, p):
        return pltpu.async_copy(table_hbm.at[idxg_v.at[li]], rows[p], gsem[p])

    def out_slice(li):
        return out_hbm.at[pl.ds(li * D, D), pl.ds(wid * BB, BB)]

    lanes = lax.iota(jnp.int32, LN)
    bidxs = [lanes + c * LN for c in range(BB // LN)]
    lanes16 = jnp.full((LN,), LN, jnp.int32)

    def transpose_scale(li, p):
        rv, tv = rows[p], trs[p]
        eoffs = []
        for c in range(BB // LN):
            xi = idx_v[li, pl.ds(c * LN, LN)]
            eoffs.append(lax.shift_left(
                jnp.bitwise_and(lax.shift_right_logical(xi, 10), 1), 6))

        @plsc.parallel_loop(0, LN, step=1, unroll=2)
        def body(d):
            ebase = lax.rem(lanes + d, lanes16)
            for e0 in range(D // LN):
                for c in range(BB // LN):
                    eidx = ebase + e0 * LN
                    g = plsc.load_gather(rv, [bidxs[c], eidx + eoffs[c]])
                    plsc.store_scatter(tv, [eidx, bidxs[c]], g * SCALE)

    gath(0, 0)
    gath(1, 1)

    def pair(k2, cr):
        for j in (0, 1):
            li = 2 * k2 + j
            p = j
            pltpu.make_async_copy(table_hbm.at[idxg_v.at[li]], rows[p],
                                  gsem[p]).wait()
            @pl.when(li >= 2)
            def _():
                pltpu.make_async_copy(trs[p], out_slice(0), ssem[p]).wait()
            transpose_scale(li, p)
            pltpu.async_copy(trs[p], out_slice(li), ssem[p])
            @pl.when(li + 2 < L)
            def _():
                gath(li + 2, p)
        return cr

    lax.fori_loop(0, L // 2, pair, 0)
    pltpu.make_async_copy(trs[0], out_slice(0), ssem[0]).wait()
    pltpu.make_async_copy(trs[1], out_slice(0), ssem[1]).wait()


def kernel(x, input_embedding_table):
    tbl2 = _pack(input_embedding_table)
    out = _gather_scale_t(x.T, tbl2)
    return out.reshape(L, D, B).transpose(2, 0, 1)


# speedup vs baseline: 1.6980x; 1.6980x over previous
import functools

import jax
import jax.numpy as jnp
from jax import lax
from jax.experimental import pallas as pl
from jax.experimental.pallas import tpu as pltpu
from jax.experimental.pallas import tpu_sc as plsc

D = 64
SCALE = 8.0
B = 4096
L = 200
BB = 128

_info = plsc.get_sparse_core_info()
NC, NS, LN = _info.num_cores, _info.num_subcores, _info.num_lanes
NW = NC * NS

_mesh = plsc.VectorSubcoreMesh(core_axis_name="c", subcore_axis_name="s")


@functools.partial(
    pl.kernel,
    mesh=_mesh,
    compiler_params=pltpu.CompilerParams(
        use_tc_tiling_on_sc=False, needs_layout_passes=False),
    out_type=jax.ShapeDtypeStruct((L, 8, B // BB, 8, BB), jnp.float32),
    scratch_types=[
        pltpu.VMEM((L, BB), jnp.int32),
        pltpu.VMEM((BB, 2 * D), jnp.float32),
        pltpu.VMEM((BB, 2 * D), jnp.float32),
        pltpu.VMEM((8, 1, 8, BB), jnp.float32),
        pltpu.VMEM((8, 1, 8, BB), jnp.float32),
        pltpu.SemaphoreType.DMA,
        pltpu.SemaphoreType.DMA,
        pltpu.SemaphoreType.DMA,
        pltpu.SemaphoreType.DMA,
    ],
)
def _gather_scale_t(xt_hbm, table_hbm, out_hbm,
                    idx_v, rows_a, rows_b, tr_a, tr_b, ga, gb, sta, stb):
    wid = lax.axis_index("s") * NC + lax.axis_index("c")
    rows = (rows_a, rows_b)
    trs = (tr_a, tr_b)
    gsem = (ga, gb)
    ssem = (sta, stb)

    pltpu.sync_copy(xt_hbm.at[:, pl.ds(wid * BB, BB)], idx_v)

    def gath(li, p):
        return pltpu.async_copy(table_hbm.at[idx_v.at[li]], rows[p], gsem[p])

    def out_slice(li):
        return out_hbm.at[li, pl.ds(0, 8), pl.ds(wid, 1)]

    lanes = lax.iota(jnp.int32, LN)
    bidxs = [lanes + c * LN for c in range(BB // LN)]
    lanes16 = jnp.full((LN,), LN, jnp.int32)
    zeros16 = jnp.zeros((LN,), jnp.int32)

    def transpose_scale(p):
        rv, tv = rows[p], trs[p]

        @plsc.parallel_loop(0, LN, step=1, unroll=2)
        def body(d):
            ebase = lax.rem(lanes + d, lanes16)
            for e0 in range(D // LN):
                eidx = ebase + e0 * LN
                kidx = lax.shift_right_logical(eidx, 3)
                sidx = jnp.bitwise_and(eidx, 7)
                for c in range(BB // LN):
                    g = plsc.load_gather(rv, [bidxs[c], eidx])
                    plsc.store_scatter(tv, [kidx, zeros16, sidx, bidxs[c]],
                                       g * SCALE)

    gath(0, 0)
    gath(1, 1)

    def pair(k2, cr):
        for j in (0, 1):
            li = 2 * k2 + j
            p = j
            pltpu.make_async_copy(table_hbm.at[idx_v.at[li]], rows[p],
                                  gsem[p]).wait()
            @pl.when(li >= 2)
            def _():
                pltpu.make_async_copy(trs[p], out_slice(0), ssem[p]).wait()
            transpose_scale(p)
            pltpu.async_copy(trs[p], out_slice(li), ssem[p])
            @pl.when(li + 2 < L)
            def _():
                gath(li + 2, p)
        return cr

    lax.fori_loop(0, L // 2, pair, 0)
    pltpu.make_async_copy(trs[0], out_slice(0), ssem[0]).wait()
    pltpu.make_async_copy(trs[1], out_slice(0), ssem[1]).wait()


def kernel(x, input_embedding_table):
    tblp = jnp.pad(input_embedding_table, ((0, 0), (0, D)))
    out = _gather_scale_t(x.T, tblp)
    return out.transpose(2, 4, 0, 1, 3).reshape(B, L, D)
